# Initial kernel scaffold; baseline (speedup 1.0000x reference)
import jax
import jax.numpy as jnp
from jax.experimental import pallas as pl


def kernel(x, w1, b1, w2, b2):
    raise NotImplementedError("write your pallas kernel here")


# speedup vs baseline: 1.1924x; 1.1924x over previous
import functools

import jax
import jax.numpy as jnp
from jax import lax
from jax.experimental import pallas as pl
from jax.experimental.pallas import tpu as pltpu

_LANE = 128
_BLOCK_BYTES = 4 * 1024 * 1024


def _round_up(n, m):
    return (n + m - 1) // m * m


def _gate_kernel(x_ref, w1_ref, b1_ref, w2t_ref, b2_ref, o_ref, *, hw_true):
    x = x_ref[...]
    nb, c, hwp = x.shape
    if hwp != hw_true:
        lane = lax.broadcasted_iota(jnp.int32, (nb, c, hwp), 2)
        valid = lane < hw_true
        x_for_sum = jnp.where(valid, x, 0.0)
        x_for_max = jnp.where(valid, x, -jnp.inf)
    else:
        x_for_sum = x
        x_for_max = x

    avg = jnp.sum(x_for_sum, axis=-1, keepdims=True) * (1.0 / hw_true)
    mx = jnp.max(x_for_max, axis=-1, keepdims=True)

    w1 = w1_ref[...][None]
    b1 = b1_ref[...][None]
    w2t = w2t_ref[...][None]
    b2 = b2_ref[...][None]

    h_avg = jnp.maximum(jnp.sum(w1 * avg, axis=1, keepdims=True) + b1, 0.0)
    h_max = jnp.maximum(jnp.sum(w1 * mx, axis=1, keepdims=True) + b1, 0.0)
    att = (jnp.sum(w2t * h_avg, axis=2, keepdims=True)
           + jnp.sum(w2t * h_max, axis=2, keepdims=True)
           + 2.0 * b2)
    o_ref[...] = x * jax.nn.sigmoid(att)


def kernel(x, w1, b1, w2, b2):
    B, C, H, W = x.shape
    HW = H * W
    Ch = w1.shape[1]
    x_flat = x.reshape(B, C, HW).astype(jnp.float32)
    w1 = w1.astype(jnp.float32)
    b1 = b1.astype(jnp.float32).reshape(1, Ch)
    w2t = w2.astype(jnp.float32).T
    b2 = b2.astype(jnp.float32).reshape(C, 1)

    HWp = _round_up(HW, _LANE)
    if HWp != HW:
        x_flat = jnp.pad(x_flat, ((0, 0), (0, 0), (0, HWp - HW)))

    nb = 1
    for cand in (4, 2, 1):
        if B % cand == 0 and cand * C * HWp * 4 <= _BLOCK_BYTES:
            nb = cand
            break

    out = pl.pallas_call(
        functools.partial(_gate_kernel, hw_true=HW),
        out_shape=jax.ShapeDtypeStruct((B, C, HWp), jnp.float32),
        grid=(B // nb,),
        in_specs=[
            pl.BlockSpec((nb, C, HWp), lambda b: (b, 0, 0)),
            pl.BlockSpec((C, Ch), lambda b: (0, 0)),
            pl.BlockSpec((1, Ch), lambda b: (0, 0)),
            pl.BlockSpec((C, Ch), lambda b: (0, 0)),
            pl.BlockSpec((C, 1), lambda b: (0, 0)),
        ],
        out_specs=pl.BlockSpec((nb, C, HWp), lambda b: (b, 0, 0)),
        compiler_params=pltpu.CompilerParams(
            dimension_semantics=("parallel",),
            vmem_limit_bytes=48 * 1024 * 1024,
        ),
    )(x_flat, w1, b1, w2t, b2)

    if HWp != HW:
        out = out[:, :, :HW]
    return out.reshape(B, C, H, W)


# speedup vs baseline: 4.1951x; 3.5184x over previous
import functools

import jax
import jax.numpy as jnp
from jax import lax
from jax.experimental import pallas as pl
from jax.experimental.pallas import tpu as pltpu

_SUBLANE = 8
_BLOCK_BYTES = 4 * 1024 * 1024


def _round_up(n, m):
    return (n + m - 1) // m * m


def _gate_kernel(x_ref, w1_ref, b1_ref, w2_ref, b2_ref, o_ref, *, hw_true):
    x = x_ref[...]
    nb, hwp, c = x.shape
    if hwp != hw_true:
        row = lax.broadcasted_iota(jnp.int32, (nb, hwp, c), 1)
        valid = row < hw_true
        x_for_sum = jnp.where(valid, x, 0.0)
        x_for_max = jnp.where(valid, x, -jnp.inf)
    else:
        x_for_sum = x
        x_for_max = x

    avg = jnp.sum(x_for_sum, axis=1) * (1.0 / hw_true)
    mx = jnp.max(x_for_max, axis=1)

    w1 = w1_ref[...]
    b1 = b1_ref[...]
    w2 = w2_ref[...]
    b2 = b2_ref[...]

    dn = (((1,), (0,)), ((), ()))
    h_a = jnp.maximum(
        lax.dot_general(avg, w1, dn, preferred_element_type=jnp.float32) + b1, 0.0)
    h_m = jnp.maximum(
        lax.dot_general(mx, w1, dn, preferred_element_type=jnp.float32) + b1, 0.0)
    att = (lax.dot_general(h_a, w2, dn, preferred_element_type=jnp.float32)
           + lax.dot_general(h_m, w2, dn, preferred_element_type=jnp.float32)
           + 2.0 * b2)
    scale = jax.nn.sigmoid(att)
    o_ref[...] = x * scale[:, None, :]


def kernel(x, w1, b1, w2, b2):
    B, C, H, W = x.shape
    HW = H * W
    Ch = w1.shape[1]

    x_nhwc = jnp.transpose(x, (0, 2, 3, 1)).reshape(B, HW, C).astype(jnp.float32)
    w1 = w1.astype(jnp.float32)
    b1 = b1.astype(jnp.float32).reshape(1, Ch)
    w2 = w2.astype(jnp.float32)
    b2 = b2.astype(jnp.float32).reshape(1, C)

    HWp = _round_up(HW, _SUBLANE)
    if HWp != HW:
        x_nhwc = jnp.pad(x_nhwc, ((0, 0), (0, HWp - HW), (0, 0)))

    nb = 1
    for cand in (4, 2, 1):
        if B % cand == 0 and cand * C * HWp * 4 <= _BLOCK_BYTES:
            nb = cand
            break

    out = pl.pallas_call(
        functools.partial(_gate_kernel, hw_true=HW),
        out_shape=jax.ShapeDtypeStruct((B, HWp, C), jnp.float32),
        grid=(B // nb,),
        in_specs=[
            pl.BlockSpec((nb, HWp, C), lambda b: (b, 0, 0)),
            pl.BlockSpec((C, Ch), lambda b: (0, 0)),
            pl.BlockSpec((1, Ch), lambda b: (0, 0)),
            pl.BlockSpec((Ch, C), lambda b: (0, 0)),
            pl.BlockSpec((1, C), lambda b: (0, 0)),
        ],
        out_specs=pl.BlockSpec((nb, HWp, C), lambda b: (b, 0, 0)),
        compiler_params=pltpu.CompilerParams(
            dimension_semantics=("parallel",),
            vmem_limit_bytes=48 * 1024 * 1024,
        ),
    )(x_nhwc, w1, b1, w2, b2)

    if HWp != HW:
        out = out[:, :HW, :]
    return jnp.transpose(out.reshape(B, H, W, C), (0, 3, 1, 2))
